# Initial kernel scaffold; baseline (speedup 1.0000x reference)
import jax
import jax.numpy as jnp
from jax.experimental import pallas as pl


def kernel(states, agent_qs, W1, b1, W2, b2, W3, b3, VW1, Vb1, VW2, Vb2):
    raise NotImplementedError("write your pallas kernel here")


# speedup vs baseline: 6.3891x; 6.3891x over previous
import functools

import jax
import jax.numpy as jnp
from jax import lax
from jax.experimental import pallas as pl

N_AGENTS = 16
N_ACTIONS = 8
SAMPLE_SIZE = 8
STATE_DIM = 64
EMBED = 64
B_TOTAL = 1024
BB = 8
GRID = B_TOTAL // BB
R = BB * SAMPLE_SIZE
ROWS = R * N_AGENTS


def _mixer_block(perm_ref, st_ref, aq_ref, w1s_ref, w1a_ref, b1_ref,
                 w2_ref, b2_ref, w3_ref, b3_ref,
                 vw1_ref, vb1_ref, vw2_ref, vb2_ref, out_ref):
    f32 = jnp.float32
    permv = perm_ref[:]
    st = st_ref[:]
    aqm = aq_ref[:]

    b_iota = lax.broadcasted_iota(jnp.int32, (BB, SAMPLE_SIZE * N_AGENTS), 0)
    gidx = permv + b_iota * N_AGENTS
    col = lax.broadcasted_iota(jnp.int32, (BB, SAMPLE_SIZE * N_AGENTS,
                                           BB * N_AGENTS), 2)
    g3 = (gidx[:, :, None] == col).astype(f32)
    gm = g3.reshape(ROWS, BB * N_AGENTS)
    p2 = jnp.dot(gm, aqm, preferred_element_type=f32)

    p2t = jnp.concatenate([p2] * N_AGENTS, axis=1)
    row_j = lax.broadcasted_iota(jnp.int32, (ROWS, 128), 0) % N_AGENTS
    col_j = lax.broadcasted_iota(jnp.int32, (ROWS, 128), 1) // N_ACTIONS
    p2e = p2t * (row_j == col_j).astype(f32)
    dsel = (lax.broadcasted_iota(jnp.int32, (R, ROWS), 1) // N_AGENTS ==
            lax.broadcasted_iota(jnp.int32, (R, ROWS), 0)).astype(f32)
    pflat = jnp.dot(dsel, p2e, preferred_element_type=f32)

    tril = (lax.broadcasted_iota(jnp.int32, (N_AGENTS, 128), 1) // N_ACTIONS
            <= lax.broadcasted_iota(jnp.int32, (N_AGENTS, 128), 0)).astype(f32)
    xmask = (pflat[:, None, :] * tril[None, :, :]).reshape(ROWS, 128)

    hpre = jnp.dot(xmask, w1a_ref[:], preferred_element_type=f32)
    sp = jnp.dot(st, w1s_ref[:], preferred_element_type=f32)
    spb = jnp.broadcast_to(sp[:, None, :],
                           (BB, SAMPLE_SIZE * N_AGENTS, EMBED)).reshape(ROWS,
                                                                        EMBED)
    h1 = jnp.maximum(hpre + spb + b1_ref[:], 0.0)
    h2 = jnp.maximum(jnp.dot(h1, w2_ref[:], preferred_element_type=f32)
                     + b2_ref[:], 0.0)
    adv = jnp.sum(h2 * w3_ref[:], axis=1, keepdims=True) + b3_ref[:]

    adv3 = adv.reshape(BB, SAMPLE_SIZE * N_AGENTS, 1)
    agent_col = lax.broadcasted_iota(jnp.int32, (BB, SAMPLE_SIZE * N_AGENTS,
                                                 N_AGENTS), 2)
    gl3 = (permv[:, :, None] == agent_col).astype(f32)
    shap = jnp.sum(adv3 * gl3, axis=1) * (1.0 / SAMPLE_SIZE)

    hv = jnp.maximum(jnp.dot(st, vw1_ref[:], preferred_element_type=f32)
                     + vb1_ref[:], 0.0)
    vs = jnp.sum(hv * vw2_ref[:], axis=1, keepdims=True) + vb2_ref[:]
    out_ref[:] = shap + vs


@functools.partial(jax.jit, static_argnums=())
def kernel(states, agent_qs, W1, b1, W2, b2, W3, b3, VW1, Vb1, VW2, Vb2):
    f32 = jnp.float32
    bs0, t = states.shape[0], states.shape[1]

    u = jax.random.uniform(jax.random.key(42), (B_TOTAL * SAMPLE_SIZE,
                                                N_AGENTS))
    pos = jnp.argsort(u, axis=1)
    perm = jnp.argsort(pos, axis=1)
    permp = perm.reshape(B_TOTAL, SAMPLE_SIZE * N_AGENTS).astype(jnp.int32)

    states_r = states.reshape(B_TOTAL, STATE_DIM)
    aq_r = agent_qs.reshape(B_TOTAL * N_AGENTS, N_ACTIONS)

    w1s = W1[:, :STATE_DIM].T
    w1a = W1[:, STATE_DIM:].T
    row = lambda v: v.reshape(1, -1).astype(f32)

    out = pl.pallas_call(
        _mixer_block,
        grid=(GRID,),
        in_specs=[
            pl.BlockSpec((BB, SAMPLE_SIZE * N_AGENTS), lambda i: (i, 0)),
            pl.BlockSpec((BB, STATE_DIM), lambda i: (i, 0)),
            pl.BlockSpec((BB * N_AGENTS, N_ACTIONS), lambda i: (i, 0)),
            pl.BlockSpec((STATE_DIM, EMBED), lambda i: (0, 0)),
            pl.BlockSpec((N_AGENTS * N_ACTIONS, EMBED), lambda i: (0, 0)),
            pl.BlockSpec((1, EMBED), lambda i: (0, 0)),
            pl.BlockSpec((EMBED, EMBED), lambda i: (0, 0)),
            pl.BlockSpec((1, EMBED), lambda i: (0, 0)),
            pl.BlockSpec((1, EMBED), lambda i: (0, 0)),
            pl.BlockSpec((1, 1), lambda i: (0, 0)),
            pl.BlockSpec((STATE_DIM, EMBED), lambda i: (0, 0)),
            pl.BlockSpec((1, EMBED), lambda i: (0, 0)),
            pl.BlockSpec((1, EMBED), lambda i: (0, 0)),
            pl.BlockSpec((1, 1), lambda i: (0, 0)),
        ],
        out_specs=pl.BlockSpec((BB, N_AGENTS), lambda i: (i, 0)),
        out_shape=jax.ShapeDtypeStruct((B_TOTAL, N_AGENTS), f32),
    )(permp, states_r, aq_r, w1s, w1a, row(b1), W2.T, row(b2), row(W3),
      jnp.reshape(b3, (1, 1)).astype(f32), VW1.T, row(Vb1), row(VW2),
      jnp.reshape(Vb2, (1, 1)).astype(f32))
    return out.reshape(bs0, t, N_AGENTS)


# speedup vs baseline: 20.1480x; 3.1535x over previous
import jax
import jax.numpy as jnp
import numpy as np
from jax import lax
from jax.experimental import pallas as pl

N_AGENTS = 16
N_ACTIONS = 8
SAMPLE_SIZE = 8
STATE_DIM = 64
EMBED = 64
B_TOTAL = 1024
BB = 16
GRID = B_TOTAL // BB
R = BB * SAMPLE_SIZE
ROWS = R * N_AGENTS
SJ = SAMPLE_SIZE * N_AGENTS

_MASKJ16 = ((np.arange(N_AGENTS)[:, None]) ==
            (np.arange(SJ)[None, :] // N_ACTIONS)).astype(np.float32)
_TRIL = ((np.arange(SJ)[None, :] // N_ACTIONS) <=
         np.arange(N_AGENTS)[:, None]).astype(np.float32)
_REPB = ((np.arange(ROWS)[:, None] // SJ) ==
         np.arange(BB)[None, :]).astype(np.float32)


def _mixer_block(gl_ref, st_ref, aq_ref, maskj_ref, tril_ref, repb_ref,
                 w1s_ref, w1a_ref, b1_ref, w2_ref, b2_ref, w3_ref, b3_ref,
                 vw1_ref, vb1_ref, vw2_ref, vb2_ref, out_ref):
    f32 = jnp.float32
    gl = gl_ref[:]
    st = st_ref[:]
    aq3 = aq_ref[:]

    aqw = jnp.concatenate([aq3] * N_AGENTS, axis=2)
    pw = lax.dot_general(gl, aqw, (((2,), (1,)), ((0,), (0,))),
                         preferred_element_type=f32)
    pw4 = pw.reshape(BB, SAMPLE_SIZE, N_AGENTS, SJ)
    pflat3 = jnp.sum(pw4 * maskj_ref[:][None, None], axis=2)

    xmask = (pflat3[:, :, None, :] *
             tril_ref[:][None, None]).reshape(ROWS, SJ)

    sp = jnp.dot(st, w1s_ref[:], preferred_element_type=f32)
    h1 = jnp.maximum(
        jnp.dot(xmask, w1a_ref[:], preferred_element_type=f32)
        + jnp.dot(repb_ref[:], sp, preferred_element_type=f32)
        + b1_ref[:], 0.0)
    h2 = jnp.maximum(jnp.dot(h1, w2_ref[:], preferred_element_type=f32)
                     + b2_ref[:], 0.0)
    adv = jnp.dot(h2, w3_ref[:], preferred_element_type=f32) + b3_ref[:]

    adv3 = adv.reshape(BB, SJ, 1)
    shap = jnp.sum(adv3 * gl, axis=1) * (1.0 / SAMPLE_SIZE)

    hv = jnp.maximum(jnp.dot(st, vw1_ref[:], preferred_element_type=f32)
                     + vb1_ref[:], 0.0)
    vs = jnp.sum(hv * vw2_ref[:], axis=1, keepdims=True) + vb2_ref[:]
    out_ref[:] = shap + vs


def kernel(states, agent_qs, W1, b1, W2, b2, W3, b3, VW1, Vb1, VW2, Vb2):
    f32 = jnp.float32
    bs0, t = states.shape[0], states.shape[1]

    u = jax.random.uniform(jax.random.key(42),
                           (B_TOTAL * SAMPLE_SIZE, N_AGENTS))
    pos = jnp.argsort(u, axis=1)
    perm = jnp.argsort(pos, axis=1).astype(jnp.int32)
    permp = perm.reshape(B_TOTAL, SJ)
    gl_const = (permp[:, :, None] ==
                jnp.arange(N_AGENTS)[None, None, :]).astype(f32)

    states_r = states.reshape(B_TOTAL, STATE_DIM)
    aq_r = agent_qs.reshape(B_TOTAL, N_AGENTS, N_ACTIONS)
    w1s = W1[:, :STATE_DIM].T
    w1a = W1[:, STATE_DIM:].T
    row = lambda v: v.reshape(1, -1).astype(f32)
    const = lambda shape: pl.BlockSpec(shape, lambda i: tuple(0 for _ in shape))

    out = pl.pallas_call(
        _mixer_block,
        grid=(GRID,),
        in_specs=[
            pl.BlockSpec((BB, SJ, N_AGENTS), lambda i: (i, 0, 0)),
            pl.BlockSpec((BB, STATE_DIM), lambda i: (i, 0)),
            pl.BlockSpec((BB, N_AGENTS, N_ACTIONS), lambda i: (i, 0, 0)),
            const((N_AGENTS, SJ)),
            const((N_AGENTS, SJ)),
            const((ROWS, BB)),
            const((STATE_DIM, EMBED)),
            const((N_AGENTS * N_ACTIONS, EMBED)),
            const((1, EMBED)),
            const((EMBED, EMBED)),
            const((1, EMBED)),
            const((EMBED, 1)),
            const((1, 1)),
            const((STATE_DIM, EMBED)),
            const((1, EMBED)),
            const((1, EMBED)),
            const((1, 1)),
        ],
        out_specs=pl.BlockSpec((BB, N_AGENTS), lambda i: (i, 0)),
        out_shape=jax.ShapeDtypeStruct((B_TOTAL, N_AGENTS), f32),
    )(gl_const, states_r, aq_r, _MASKJ16, _TRIL, _REPB, w1s, w1a, row(b1),
      W2.T, row(b2), W3.reshape(-1, 1).astype(f32),
      jnp.reshape(b3, (1, 1)).astype(f32), VW1.T, row(Vb1), row(VW2),
      jnp.reshape(Vb2, (1, 1)).astype(f32))
    return out.reshape(bs0, t, N_AGENTS)
